# baseline (speedup 1.0000x reference)
import functools

import jax
import jax.numpy as jnp
from jax import lax
from jax.experimental import pallas as pl
from jax.experimental.pallas import tpu as pltpu
from jax.experimental.pallas import tpu_sc as plsc

HIDDEN = 768
EPS = 1e-5
L = 16
NC, NS = 2, 16
NW = NC * NS
CHUNK = 16
NBUF = 4
NGROUP = CHUNK // L
COLS_PER_STEP = 16
NACC = 4
STORE_BYTES = CHUNK * HIDDEN * 4


def _rsqrt16(x):
    i = lax.bitcast_convert_type(x, jnp.int32)
    i = jnp.int32(0x5F3759DF) - lax.shift_right_logical(i, jnp.int32(1))
    y = lax.bitcast_convert_type(i, jnp.float32)
    for _ in range(3):
        y = y * (jnp.float32(1.5) - jnp.float32(0.5) * x * y * y)
    return y


def _full16(val, dtype=jnp.float32):
    return jnp.full((L,), val, dtype)


def _allsum16(x):
    lanes = lax.iota(jnp.int32, L)
    dnums = lax.GatherDimensionNumbers(
        offset_dims=(), collapsed_slice_dims=(0,), start_index_map=(0,))
    for k in (1, 2, 4, 8):
        idx = (lanes ^ k).reshape(L, 1)
        x = x + lax.gather(x, idx, dnums, slice_sizes=(1,),
                           mode=lax.GatherScatterMode.PROMISE_IN_BOUNDS)
    return x


GROUP = 8
NSLICE = HIDDEN // L


def _normalize_chunk(in_v, out_v, w_v, b):
    inv_h = jnp.float32(1.0 / HIDDEN)
    lanes = lax.iota(jnp.int32, L)
    zero = jnp.zeros((L,), jnp.float32)

    for g in range(CHUNK // GROUP):
        r0 = g * GROUP

        def pa_body(j, carry):
            accs = list(carry)
            for r in range(GROUP):
                v = in_v[b, r0 + r, pl.ds(j * L, L)]
                accs[r] = accs[r] + v
                accs[GROUP + r] = accs[GROUP + r] + v * v
            return tuple(accs)

        accs = lax.fori_loop(0, NSLICE, pa_body, (zero,) * (2 * GROUP))

        sums, sumsq = zero, zero
        for r in range(GROUP):
            st = _allsum16(accs[r])
            qt = _allsum16(accs[GROUP + r])
            sums = jnp.where(lanes == r, st, sums)
            sumsq = jnp.where(lanes == r, qt, sumsq)
        mean8 = sums * inv_h
        var8 = sumsq * inv_h - mean8 * mean8
        rstd8 = _rsqrt16(var8 + jnp.float32(EPS))
        shift8 = mean8 * rstd8
        a_r = [_full16(rstd8[r]) for r in range(GROUP)]
        b_r = [_full16(shift8[r]) for r in range(GROUP)]

        def pc_body(j, carry):
            w_j = w_v[pl.ds(j * L, L)]
            for r in range(GROUP):
                x = in_v[b, r0 + r, pl.ds(j * L, L)]
                out_v[b, r0 + r, pl.ds(j * L, L)] = (x * a_r[r] - b_r[r]) * w_j
            return carry

        lax.fori_loop(0, NSLICE, pc_body, 0)


def _build_sc_kernel(B):
    b_per_w = B // NW
    n_chunks = b_per_w // CHUNK
    mesh = plsc.VectorSubcoreMesh(core_axis_name="c", subcore_axis_name="s")

    @functools.partial(
        pl.kernel,
        mesh=mesh,
        compiler_params=pltpu.CompilerParams(
            use_tc_tiling_on_sc=False, needs_layout_passes=False),
        out_type=jax.ShapeDtypeStruct((B, HIDDEN), jnp.float32),
        scratch_types=[
            pltpu.VMEM((n_chunks, CHUNK), jnp.int32),
            pltpu.VMEM((NBUF, CHUNK, HIDDEN), jnp.float32),
            pltpu.VMEM((NBUF, CHUNK, HIDDEN), jnp.float32),
            pltpu.VMEM((HIDDEN,), jnp.float32),
        ] + [pltpu.SemaphoreType.DMA] * (2 * NBUF),
    )
    def k(ids_hbm, table_hbm, w_hbm, out_hbm,
          idx_v, in_v, out_v, w_v, *sems):
        gsems = sems[:NBUF]
        ssems = sems[NBUF:]
        wid = lax.axis_index("s") * NC + lax.axis_index("c")
        base = wid * b_per_w
        pltpu.sync_copy(w_hbm, w_v)
        pltpu.sync_copy(ids_hbm.at[wid], idx_v)

        def issue_gather(ci, buf):
            idx_vec = idx_v[ci, pl.ds(0, CHUNK)]
            for r in range(CHUNK):
                pltpu.async_copy(table_hbm.at[pl.ds(idx_vec[r], 1)],
                                 in_v.at[buf, pl.ds(r, 1)],
                                 gsems[buf])

        def wait_gather(ci, buf):
            pltpu.make_async_copy(table_hbm.at[pl.ds(0, CHUNK)],
                                  in_v.at[buf],
                                  gsems[buf]).wait()

        def issue_store(ci, buf):
            pltpu.async_copy(out_v.at[buf],
                             out_hbm.at[pl.ds(base + ci * CHUNK, CHUNK)],
                             ssems[buf])

        def wait_store(buf):
            pltpu.make_async_copy(out_v.at[buf],
                                  out_hbm.at[pl.ds(base, CHUNK)],
                                  ssems[buf]).wait()

        for b in range(NBUF):
            issue_gather(b, b)

        for b in range(NBUF):
            wait_gather(b, b)
            _normalize_chunk(in_v, out_v, w_v, b)
            issue_store(b, b)
            issue_gather(b + NBUF, b)

        def chunk_round(ci2, _):
            for b in range(NBUF):
                ci = ci2 * NBUF + b
                wait_gather(ci, b)
                wait_store(b)
                _normalize_chunk(in_v, out_v, w_v, b)
                issue_store(ci, b)
                @pl.when(ci + NBUF < n_chunks)
                def _():
                    issue_gather(ci + NBUF, b)
            return 0

        lax.fori_loop(1, n_chunks // NBUF, chunk_round, 0)
        for b in range(NBUF):
            wait_store(b)

    return k


TC_ROWS = 256


def _tc_embed_ln(ids_flat, tok_embeddings, norm_weight):
    N = ids_flat.shape[0]
    n_blocks = N // TC_ROWS

    def body(idx_ref, table_ref, w_ref, out_ref, buf, sem0, sem1):
        i = pl.program_id(0)
        sems = (sem0, sem1)

        def fire(block, b):
            base = block * TC_ROWS
            for r in range(TC_ROWS):
                pltpu.make_async_copy(
                    table_ref.at[pl.ds(idx_ref[base + r], 1)],
                    buf.at[b, pl.ds(r, 1)], sems[b]).start()

        def drain(b):
            pltpu.make_async_copy(
                table_ref.at[pl.ds(0, TC_ROWS)], buf.at[b], sems[b]).wait()

        @pl.when(i == 0)
        def _():
            fire(0, 0)

        for par in (0, 1):
            @pl.when(jnp.logical_and(i + 1 < n_blocks, (i + 1) % 2 == par))
            def _(par=par):
                fire(i + 1, par)

        b = i % 2

        for par in (0, 1):
            @pl.when(b == par)
            def _(par=par):
                drain(par)

        x = buf[pl.ds(b, 1)][0]
        mean = jnp.mean(x, axis=-1, keepdims=True)
        var = jnp.mean(x * x, axis=-1, keepdims=True) - mean * mean
        rstd = jax.lax.rsqrt(var + jnp.float32(EPS))
        out_ref[...] = (x - mean) * rstd * w_ref[...]

    grid_spec = pltpu.PrefetchScalarGridSpec(
        num_scalar_prefetch=1,
        grid=(n_blocks,),
        in_specs=[
            pl.BlockSpec(memory_space=pl.ANY),
            pl.BlockSpec((HIDDEN,), lambda i, idx: (0,)),
        ],
        out_specs=pl.BlockSpec((TC_ROWS, HIDDEN), lambda i, idx: (i, 0)),
        scratch_shapes=[
            pltpu.VMEM((2, TC_ROWS, HIDDEN), jnp.float32),
            pltpu.SemaphoreType.DMA,
            pltpu.SemaphoreType.DMA,
        ],
    )
    return pl.pallas_call(
        body,
        grid_spec=grid_spec,
        out_shape=jax.ShapeDtypeStruct((N, HIDDEN), jnp.float32),
    )(ids_flat, tok_embeddings, norm_weight)


SC_TOKENS = 10240


@jax.jit
def kernel(input_ids, tok_embeddings, norm_weight):
    B_, S_ = input_ids.shape
    B = B_ * S_
    ids_flat = input_ids.astype(jnp.int32).reshape(B)
    if SC_TOKENS == 0:
        out = _tc_embed_ln(ids_flat, tok_embeddings, norm_weight)
    elif SC_TOKENS == B:
        ids3 = ids_flat.reshape(NW, (B // NW) // CHUNK, CHUNK)
        out = _build_sc_kernel(B)(ids3, tok_embeddings, norm_weight)
    else:
        ids_sc = ids_flat[:SC_TOKENS].reshape(
            NW, (SC_TOKENS // NW) // CHUNK, CHUNK)
        out_sc = _build_sc_kernel(SC_TOKENS)(ids_sc, tok_embeddings,
                                             norm_weight)
        out_tc = _tc_embed_ln(ids_flat[SC_TOKENS:], tok_embeddings,
                              norm_weight)
        out = jnp.concatenate([out_sc, out_tc], axis=0)
    return out.reshape(B_, S_, HIDDEN)


# speedup vs baseline: 2.4870x; 2.4870x over previous
import functools

import jax
import jax.numpy as jnp
from jax import lax
from jax.experimental import pallas as pl
from jax.experimental.pallas import tpu as pltpu
from jax.experimental.pallas import tpu_sc as plsc

HIDDEN = 768
EPS = 1e-5
L = 16
NC, NS = 2, 16
NW = NC * NS
CHUNK = 16
NBUF = 4
NGROUP = CHUNK // L
COLS_PER_STEP = 16
NACC = 4
STORE_BYTES = CHUNK * HIDDEN * 4


def _rsqrt16(x):
    i = lax.bitcast_convert_type(x, jnp.int32)
    i = jnp.int32(0x5F3759DF) - lax.shift_right_logical(i, jnp.int32(1))
    y = lax.bitcast_convert_type(i, jnp.float32)
    for _ in range(3):
        y = y * (jnp.float32(1.5) - jnp.float32(0.5) * x * y * y)
    return y


def _full16(val, dtype=jnp.float32):
    return jnp.full((L,), val, dtype)


def _allsum16(x):
    lanes = lax.iota(jnp.int32, L)
    dnums = lax.GatherDimensionNumbers(
        offset_dims=(), collapsed_slice_dims=(0,), start_index_map=(0,))
    for k in (1, 2, 4, 8):
        idx = (lanes ^ k).reshape(L, 1)
        x = x + lax.gather(x, idx, dnums, slice_sizes=(1,),
                           mode=lax.GatherScatterMode.PROMISE_IN_BOUNDS)
    return x


GROUP = 8
NSLICE = HIDDEN // L


def _normalize_chunk(in_v, out_v, w_v, b):
    inv_h = jnp.float32(1.0 / HIDDEN)
    lanes = lax.iota(jnp.int32, L)
    zero = jnp.zeros((L,), jnp.float32)

    for g in range(CHUNK // GROUP):
        r0 = g * GROUP

        def pa_body(j, carry):
            accs = list(carry)
            for r in range(GROUP):
                v = in_v[b, r0 + r, pl.ds(j * L, L)]
                accs[r] = accs[r] + v
                accs[GROUP + r] = accs[GROUP + r] + v * v
            return tuple(accs)

        accs = lax.fori_loop(0, NSLICE, pa_body, (zero,) * (2 * GROUP))

        sums, sumsq = zero, zero
        for r in range(GROUP):
            st = _allsum16(accs[r])
            qt = _allsum16(accs[GROUP + r])
            sums = jnp.where(lanes == r, st, sums)
            sumsq = jnp.where(lanes == r, qt, sumsq)
        mean8 = sums * inv_h
        var8 = sumsq * inv_h - mean8 * mean8
        rstd8 = _rsqrt16(var8 + jnp.float32(EPS))
        shift8 = mean8 * rstd8
        a_r = [_full16(rstd8[r]) for r in range(GROUP)]
        b_r = [_full16(shift8[r]) for r in range(GROUP)]

        def pc_body(j, carry):
            w_j = w_v[pl.ds(j * L, L)]
            for r in range(GROUP):
                x = in_v[b, r0 + r, pl.ds(j * L, L)]
                out_v[b, r0 + r, pl.ds(j * L, L)] = (x * a_r[r] - b_r[r]) * w_j
            return carry

        lax.fori_loop(0, NSLICE, pc_body, 0)


def _build_sc_kernel(B):
    b_per_w = B // NW
    n_chunks = b_per_w // CHUNK
    mesh = plsc.VectorSubcoreMesh(core_axis_name="c", subcore_axis_name="s")

    @functools.partial(
        pl.kernel,
        mesh=mesh,
        compiler_params=pltpu.CompilerParams(
            use_tc_tiling_on_sc=False, needs_layout_passes=False),
        out_type=jax.ShapeDtypeStruct((B, HIDDEN), jnp.float32),
        scratch_types=[
            pltpu.VMEM((n_chunks, CHUNK), jnp.int32),
            pltpu.VMEM((NBUF, CHUNK, HIDDEN), jnp.float32),
            pltpu.VMEM((NBUF, CHUNK, HIDDEN), jnp.float32),
            pltpu.VMEM((HIDDEN,), jnp.float32),
        ] + [pltpu.SemaphoreType.DMA] * (2 * NBUF),
    )
    def k(ids_hbm, table_hbm, w_hbm, out_hbm,
          idx_v, in_v, out_v, w_v, *sems):
        gsems = sems[:NBUF]
        ssems = sems[NBUF:]
        wid = lax.axis_index("s") * NC + lax.axis_index("c")
        base = wid * b_per_w
        pltpu.sync_copy(w_hbm, w_v)
        pltpu.sync_copy(ids_hbm.at[wid], idx_v)

        def issue_gather(ci, buf):
            idx_vec = idx_v[ci, pl.ds(0, CHUNK)]
            for r in range(CHUNK):
                pltpu.async_copy(table_hbm.at[pl.ds(idx_vec[r], 1)],
                                 in_v.at[buf, pl.ds(r, 1)],
                                 gsems[buf])

        def wait_gather(ci, buf):
            pltpu.make_async_copy(table_hbm.at[pl.ds(0, CHUNK)],
                                  in_v.at[buf],
                                  gsems[buf]).wait()

        def issue_store(ci, buf):
            pltpu.async_copy(out_v.at[buf],
                             out_hbm.at[pl.ds(base + ci * CHUNK, CHUNK)],
                             ssems[buf])

        def wait_store(buf):
            pltpu.make_async_copy(out_v.at[buf],
                                  out_hbm.at[pl.ds(base, CHUNK)],
                                  ssems[buf]).wait()

        for b in range(NBUF):
            issue_gather(b, b)

        for b in range(NBUF):
            wait_gather(b, b)
            _normalize_chunk(in_v, out_v, w_v, b)
            issue_store(b, b)
            issue_gather(b + NBUF, b)

        def chunk_round(ci2, _):
            for b in range(NBUF):
                ci = ci2 * NBUF + b
                wait_gather(ci, b)
                wait_store(b)
                _normalize_chunk(in_v, out_v, w_v, b)
                issue_store(ci, b)
                @pl.when(ci + NBUF < n_chunks)
                def _():
                    issue_gather(ci + NBUF, b)
            return 0

        lax.fori_loop(1, n_chunks // NBUF, chunk_round, 0)
        for b in range(NBUF):
            wait_store(b)

    return k


TC_ROWS = 512


def _tc_embed_ln(ids_flat, tok_embeddings, norm_weight):
    N = ids_flat.shape[0]
    n_blocks = N // TC_ROWS

    def body(idx_ref, table_ref, w_ref, out_ref, buf, sem0, sem1):
        i = pl.program_id(0)
        sems = (sem0, sem1)

        def fire(block, b):
            base = block * TC_ROWS
            for r in range(TC_ROWS):
                pltpu.make_async_copy(
                    table_ref.at[pl.ds(idx_ref[base + r], 1)],
                    buf.at[b, pl.ds(r, 1)], sems[b]).start()

        def drain(b):
            pltpu.make_async_copy(
                table_ref.at[pl.ds(0, TC_ROWS)], buf.at[b], sems[b]).wait()

        @pl.when(i == 0)
        def _():
            fire(0, 0)

        for par in (0, 1):
            @pl.when(jnp.logical_and(i + 1 < n_blocks, (i + 1) % 2 == par))
            def _(par=par):
                fire(i + 1, par)

        b = i % 2

        for par in (0, 1):
            @pl.when(b == par)
            def _(par=par):
                drain(par)

        x = buf[pl.ds(b, 1)][0]
        mean = jnp.mean(x, axis=-1, keepdims=True)
        var = jnp.mean(x * x, axis=-1, keepdims=True) - mean * mean
        rstd = jax.lax.rsqrt(var + jnp.float32(EPS))
        out_ref[...] = (x - mean) * rstd * w_ref[...]

    grid_spec = pltpu.PrefetchScalarGridSpec(
        num_scalar_prefetch=1,
        grid=(n_blocks,),
        in_specs=[
            pl.BlockSpec(memory_space=pl.ANY),
            pl.BlockSpec((HIDDEN,), lambda i, idx: (0,)),
        ],
        out_specs=pl.BlockSpec((TC_ROWS, HIDDEN), lambda i, idx: (i, 0)),
        scratch_shapes=[
            pltpu.VMEM((2, TC_ROWS, HIDDEN), jnp.float32),
            pltpu.SemaphoreType.DMA,
            pltpu.SemaphoreType.DMA,
        ],
    )
    return pl.pallas_call(
        body,
        grid_spec=grid_spec,
        out_shape=jax.ShapeDtypeStruct((N, HIDDEN), jnp.float32),
    )(ids_flat, tok_embeddings, norm_weight)


SC_TOKENS = 0


@jax.jit
def kernel(input_ids, tok_embeddings, norm_weight):
    B_, S_ = input_ids.shape
    B = B_ * S_
    ids_flat = input_ids.astype(jnp.int32).reshape(B)
    if SC_TOKENS == 0:
        out = _tc_embed_ln(ids_flat, tok_embeddings, norm_weight)
    elif SC_TOKENS == B:
        ids3 = ids_flat.reshape(NW, (B // NW) // CHUNK, CHUNK)
        out = _build_sc_kernel(B)(ids3, tok_embeddings, norm_weight)
    else:
        ids_sc = ids_flat[:SC_TOKENS].reshape(
            NW, (SC_TOKENS // NW) // CHUNK, CHUNK)
        out_sc = _build_sc_kernel(SC_TOKENS)(ids_sc, tok_embeddings,
                                             norm_weight)
        out_tc = _tc_embed_ln(ids_flat[SC_TOKENS:], tok_embeddings,
                              norm_weight)
        out = jnp.concatenate([out_sc, out_tc], axis=0)
    return out.reshape(B_, S_, HIDDEN)
